# baseline (speedup 1.0000x reference)
import functools

import jax
import jax.numpy as jnp
from jax.experimental import pallas as pl
from jax.experimental.pallas import tpu as pltpu

_LANES = 512
_BLOCK_ROWS = 4096
_CHUNK = 128
_N_LEVELS = 3


def _shift_cols(x, s):
  return jnp.concatenate([x[:, s:], x[:, :s]], axis=1)


def _shift_rows(x, s):
  return jnp.concatenate([x[s:, :], x[:s, :]], axis=0)


def _tile_accumulate(v, accs):
  out = list(accs)
  out[0] = accs[0] + jnp.abs(v)
  x = v
  for lvl, s in enumerate((1, 2, 4)):
    rl = _shift_cols(x, s)
    cs = x + rl
    cd = x - rl
    css = _shift_rows(cs, s)
    acd = jnp.abs(cd)
    acds = _shift_rows(acd, s)
    out[1 + 2 * lvl] = accs[1 + 2 * lvl] + jnp.abs(cs - css)
    out[2 + 2 * lvl] = accs[2 + 2 * lvl] + jnp.maximum(acd, acds)
    if s < 4:
      x = cs + css
  return out


def _wavelet_kernel(inv_n, p_ref, t_ref, o_ref):
  def body(it, accs):
    base_row = it * _CHUNK
    for r in range(_CHUNK // 8):
      for c in range(_LANES // 128):
        rows = pl.ds(base_row + r * 8, 8)
        cols = slice(c * 128, (c + 1) * 128)
        v = p_ref[rows, cols] - t_ref[rows, cols]
        accs = _tile_accumulate(v, accs)
    return accs

  zero = jnp.zeros((8, 128), jnp.float32)
  accs = jax.lax.fori_loop(0, _BLOCK_ROWS // _CHUNK, body,
                           [zero] * (1 + 2 * _N_LEVELS))

  row = jax.lax.broadcasted_iota(jnp.int32, (8, 128), 0)
  col = jax.lax.broadcasted_iota(jnp.int32, (8, 128), 1)
  total = accs[0] * jnp.float32(inv_n)
  for lvl in range(_N_LEVELS):
    k = lvl + 1
    m = (1 << k) - 1
    mask = ((row & m) == 0) & ((col & m) == 0)
    wk = jnp.float32((2.0 ** k) * inv_n)
    lvl_sum = accs[1 + 2 * lvl] + 2.0 * accs[2 + 2 * lvl]
    total = total + jnp.where(mask, wk, jnp.float32(0.0)) * lvl_sum
  o_ref[0] = total


def _dwt_partials(p, t, inv_n):
  par = p.shape[0] // _BLOCK_ROWS
  body = functools.partial(_wavelet_kernel, inv_n)
  return pl.pallas_call(
      body,
      grid=(par,),
      in_specs=[
          pl.BlockSpec((_BLOCK_ROWS, _LANES), lambda i: (i, 0)),
          pl.BlockSpec((_BLOCK_ROWS, _LANES), lambda i: (i, 0)),
      ],
      out_specs=pl.BlockSpec((1, 8, 128), lambda i: (i, 0, 0)),
      out_shape=jax.ShapeDtypeStruct((par, 8, 128), jnp.float32),
      compiler_params=pltpu.CompilerParams(
          dimension_semantics=("parallel",),
      ),
  )(p, t)


@jax.jit
def kernel(pred, target):
  n_total = pred.size
  p = pred.reshape(-1, _LANES)
  t = target.reshape(-1, _LANES)
  out = _dwt_partials(p, t, 1.0 / n_total)
  return jnp.sum(out)


# speedup vs baseline: 1.0149x; 1.0149x over previous
import functools

import jax
import jax.numpy as jnp
from jax.experimental import pallas as pl
from jax.experimental.pallas import tpu as pltpu

_LANES = 512
_BLOCK_ROWS = 2048
_CHUNK = 128
_N_LEVELS = 3


def _shift_cols(x, s):
  return jnp.concatenate([x[:, s:], x[:, :s]], axis=1)


def _shift_rows(x, s):
  return jnp.concatenate([x[s:, :], x[:s, :]], axis=0)


def _tile_accumulate(v, accs):
  out = list(accs)
  out[0] = accs[0] + jnp.abs(v)
  x = v
  for lvl, s in enumerate((1, 2, 4)):
    rl = _shift_cols(x, s)
    cs = x + rl
    cd = x - rl
    css = _shift_rows(cs, s)
    acd = jnp.abs(cd)
    acds = _shift_rows(acd, s)
    out[1 + 2 * lvl] = accs[1 + 2 * lvl] + jnp.abs(cs - css)
    out[2 + 2 * lvl] = accs[2 + 2 * lvl] + jnp.maximum(acd, acds)
    if s < 4:
      x = cs + css
  return out


def _wavelet_kernel(inv_n, p_ref, t_ref, o_ref):
  def body(it, accs):
    base_row = it * _CHUNK
    for r in range(_CHUNK // 8):
      for c in range(_LANES // 128):
        rows = pl.ds(base_row + r * 8, 8)
        cols = slice(c * 128, (c + 1) * 128)
        v = p_ref[rows, cols] - t_ref[rows, cols]
        accs = _tile_accumulate(v, accs)
    return accs

  zero = jnp.zeros((8, 128), jnp.float32)
  accs = jax.lax.fori_loop(0, _BLOCK_ROWS // _CHUNK, body,
                           [zero] * (1 + 2 * _N_LEVELS))

  row = jax.lax.broadcasted_iota(jnp.int32, (8, 128), 0)
  col = jax.lax.broadcasted_iota(jnp.int32, (8, 128), 1)
  total = accs[0] * jnp.float32(inv_n)
  for lvl in range(_N_LEVELS):
    k = lvl + 1
    m = (1 << k) - 1
    mask = ((row & m) == 0) & ((col & m) == 0)
    wk = jnp.float32((2.0 ** k) * inv_n)
    lvl_sum = accs[1 + 2 * lvl] + 2.0 * accs[2 + 2 * lvl]
    total = total + jnp.where(mask, wk, jnp.float32(0.0)) * lvl_sum
  o_ref[0] = total


def _dwt_partials(p, t, inv_n):
  par = p.shape[0] // _BLOCK_ROWS
  body = functools.partial(_wavelet_kernel, inv_n)
  return pl.pallas_call(
      body,
      grid=(par,),
      in_specs=[
          pl.BlockSpec((_BLOCK_ROWS, _LANES), lambda i: (i, 0)),
          pl.BlockSpec((_BLOCK_ROWS, _LANES), lambda i: (i, 0)),
      ],
      out_specs=pl.BlockSpec((1, 8, 128), lambda i: (i, 0, 0)),
      out_shape=jax.ShapeDtypeStruct((par, 8, 128), jnp.float32),
      compiler_params=pltpu.CompilerParams(
          dimension_semantics=("parallel",),
      ),
  )(p, t)


@jax.jit
def kernel(pred, target):
  n_total = pred.size
  p = pred.reshape(-1, _LANES)
  t = target.reshape(-1, _LANES)
  out = _dwt_partials(p, t, 1.0 / n_total)
  return jnp.sum(out)
